# Initial kernel scaffold; baseline (speedup 1.0000x reference)
import jax
import jax.numpy as jnp
from jax.experimental import pallas as pl


def kernel(coords, row_embed, col_embed, W, b):
    raise NotImplementedError("write your pallas kernel here")


# speedup vs baseline: 4.4148x; 4.4148x over previous
import functools

import jax
import jax.numpy as jnp
from jax import lax
from jax.experimental import pallas as pl
from jax.experimental.pallas import tpu as pltpu
from jax.experimental.pallas import tpu_sc as plsc

D_OUT = 128
HALF = 64
GRID = 256
NC, NS = 2, 16
NW = NC * NS
SPOTS = 16 * 4096
PER_W = SPOTS // NW
CHUNK = 128
NCHUNK = PER_W // CHUNK


def _tables_body(row_ref, col_ref, w_ref, b_ref, tr_ref, tc_ref):
    w = w_ref[...]
    tr = lax.dot_general(row_ref[...], w[:, :HALF],
                         (((1,), (1,)), ((), ())),
                         preferred_element_type=jnp.float32)
    tr_ref[...] = tr + b_ref[...]
    tc_ref[...] = lax.dot_general(col_ref[...], w[:, HALF:],
                                  (((1,), (1,)), ((), ())),
                                  preferred_element_type=jnp.float32)


def _make_tables(row_embed, col_embed, w, b):
    return pl.pallas_call(
        _tables_body,
        out_shape=(
            jax.ShapeDtypeStruct((GRID, D_OUT), jnp.float32),
            jax.ShapeDtypeStruct((GRID, D_OUT), jnp.float32),
        ),
    )(row_embed, col_embed, w, b.reshape(1, D_OUT))


_sc_mesh = plsc.VectorSubcoreMesh(core_axis_name="c", subcore_axis_name="s")


@functools.partial(
    pl.kernel,
    out_type=jax.ShapeDtypeStruct((SPOTS, D_OUT), jnp.float32),
    mesh=_sc_mesh,
    scratch_types=[
        pltpu.VMEM((NCHUNK, CHUNK), jnp.int32),
        pltpu.VMEM((NCHUNK, CHUNK), jnp.int32),
        pltpu.VMEM((CHUNK, D_OUT), jnp.float32),
        pltpu.VMEM((CHUNK, D_OUT), jnp.float32),
        pltpu.SemaphoreType.DMA,
        pltpu.SemaphoreType.DMA,
    ],
)
def _sc_lookup(tr_hbm, tc_hbm, rows_hbm, cols_hbm, out_hbm,
               idxr, idxc, bufr, bufc, semr, semc):
    wid = lax.axis_index("s") * NC + lax.axis_index("c")
    pltpu.sync_copy(rows_hbm.at[wid], idxr)
    pltpu.sync_copy(cols_hbm.at[wid], idxc)
    for j in range(NCHUNK):
        cpr = pltpu.async_copy(tr_hbm.at[idxr.at[j]], bufr, semr)
        cpc = pltpu.async_copy(tc_hbm.at[idxc.at[j]], bufc, semc)
        cpr.wait()
        cpc.wait()

        def _add_row(r, _):
            for c in range(D_OUT // 16):
                sl = pl.ds(c * 16, 16)
                bufr[r, sl] = bufr[r, sl] + bufc[r, sl]
            return 0

        lax.fori_loop(0, CHUNK, _add_row, 0)
        pltpu.sync_copy(bufr, out_hbm.at[pl.ds(wid * PER_W + j * CHUNK, CHUNK)])


def kernel(coords, row_embed, col_embed, W, b):
    batch, n_spots, _ = coords.shape
    tr, tc = _make_tables(row_embed, col_embed, W, b)
    cc = jnp.clip(coords.astype(jnp.int32), 0, GRID - 1)
    rows = cc[..., 0].reshape(NW, NCHUNK, CHUNK)
    cols = cc[..., 1].reshape(NW, NCHUNK, CHUNK)
    out = _sc_lookup(tr, tc, rows, cols)
    return out.reshape(batch, n_spots, D_OUT)


# speedup vs baseline: 4.5875x; 1.0391x over previous
import functools

import jax
import jax.numpy as jnp
from jax import lax
from jax.experimental import pallas as pl
from jax.experimental.pallas import tpu as pltpu
from jax.experimental.pallas import tpu_sc as plsc

D_OUT = 128
HALF = 64
GRID = 256
NC, NS = 2, 16
NW = NC * NS
SPOTS = 16 * 4096
PER_W = SPOTS // NW
CHUNK = 128
NCHUNK = PER_W // CHUNK


def _tables_body(row_ref, col_ref, w_ref, b_ref, tr_ref, tc_ref):
    w = w_ref[...]
    tr = lax.dot_general(row_ref[...], w[:, :HALF],
                         (((1,), (1,)), ((), ())),
                         preferred_element_type=jnp.float32)
    tr_ref[...] = tr + b_ref[...]
    tc_ref[...] = lax.dot_general(col_ref[...], w[:, HALF:],
                                  (((1,), (1,)), ((), ())),
                                  preferred_element_type=jnp.float32)


def _make_tables(row_embed, col_embed, w, b):
    return pl.pallas_call(
        _tables_body,
        out_shape=(
            jax.ShapeDtypeStruct((GRID, D_OUT), jnp.float32),
            jax.ShapeDtypeStruct((GRID, D_OUT), jnp.float32),
        ),
    )(row_embed, col_embed, w, b.reshape(1, D_OUT))


_sc_mesh = plsc.VectorSubcoreMesh(core_axis_name="c", subcore_axis_name="s")


@functools.partial(
    pl.kernel,
    out_type=jax.ShapeDtypeStruct((SPOTS, D_OUT), jnp.float32),
    mesh=_sc_mesh,
    scratch_types=[
        pltpu.VMEM((NCHUNK, CHUNK), jnp.int32),
        pltpu.VMEM((NCHUNK, CHUNK), jnp.int32),
        pltpu.VMEM((CHUNK, D_OUT), jnp.float32),
        pltpu.VMEM((CHUNK, D_OUT), jnp.float32),
        pltpu.VMEM((CHUNK, D_OUT), jnp.float32),
        pltpu.VMEM((CHUNK, D_OUT), jnp.float32),
        pltpu.VMEM((CHUNK, D_OUT), jnp.float32),
        pltpu.VMEM((CHUNK, D_OUT), jnp.float32),
        pltpu.SemaphoreType.DMA,
        pltpu.SemaphoreType.DMA,
        pltpu.SemaphoreType.DMA,
        pltpu.SemaphoreType.DMA,
        pltpu.SemaphoreType.DMA,
        pltpu.SemaphoreType.DMA,
    ],
)
def _sc_lookup(tr_hbm, tc_hbm, rows_hbm, cols_hbm, out_hbm,
               idxr, idxc, bufr0, bufr1, bufc0, bufc1, bufo0, bufo1,
               semr0, semr1, semc0, semc1, semo0, semo1):
    wid = lax.axis_index("s") * NC + lax.axis_index("c")
    base = wid * PER_W
    bufr = (bufr0, bufr1)
    bufc = (bufc0, bufc1)
    bufo = (bufo0, bufo1)
    semr = (semr0, semr1)
    semc = (semc0, semc1)
    semo = (semo0, semo1)
    pltpu.sync_copy(rows_hbm.at[wid], idxr)
    pltpu.sync_copy(cols_hbm.at[wid], idxc)
    gr = [pltpu.async_copy(tr_hbm.at[idxr.at[p]], bufr[p], semr[p])
          for p in range(2)]
    gc = [pltpu.async_copy(tc_hbm.at[idxc.at[p]], bufc[p], semc[p])
          for p in range(2)]
    oc = [None, None]
    for j in range(NCHUNK):
        p = j & 1
        gr[p].wait()
        gc[p].wait()
        if oc[p] is not None:
            oc[p].wait()

        def _add_row(r, _, p=p):
            for c in range(D_OUT // 16):
                sl = pl.ds(c * 16, 16)
                bufo[p][r, sl] = bufr[p][r, sl] + bufc[p][r, sl]
            return 0

        lax.fori_loop(0, CHUNK, _add_row, 0)
        oc[p] = pltpu.async_copy(
            bufo[p], out_hbm.at[pl.ds(base + j * CHUNK, CHUNK)], semo[p])
        if j + 2 < NCHUNK:
            gr[p] = pltpu.async_copy(tr_hbm.at[idxr.at[j + 2]], bufr[p], semr[p])
            gc[p] = pltpu.async_copy(tc_hbm.at[idxc.at[j + 2]], bufc[p], semc[p])
    oc[0].wait()
    oc[1].wait()


def kernel(coords, row_embed, col_embed, W, b):
    batch, n_spots, _ = coords.shape
    tr, tc = _make_tables(row_embed, col_embed, W, b)
    cc = jnp.clip(coords.astype(jnp.int32), 0, GRID - 1)
    rows = cc[..., 0].reshape(NW, NCHUNK, CHUNK)
    cols = cc[..., 1].reshape(NW, NCHUNK, CHUNK)
    out = _sc_lookup(tr, tc, rows, cols)
    return out.reshape(batch, n_spots, D_OUT)


# speedup vs baseline: 8.4929x; 1.8513x over previous
import functools

import jax
import jax.numpy as jnp
from jax import lax
from jax.experimental import pallas as pl
from jax.experimental.pallas import tpu as pltpu
from jax.experimental.pallas import tpu_sc as plsc

D_OUT = 128
HALF = 64
GRID = 256
NC, NS = 2, 16
NW = NC * NS
SPOTS = 16 * 4096
PER_W = SPOTS // NW
CHUNK = 128
NCHUNK = PER_W // CHUNK


def _tables_body(row_ref, col_ref, w_ref, b_ref, tr_ref, tc_ref):
    w = w_ref[...]
    tr = lax.dot_general(row_ref[...], w[:, :HALF],
                         (((1,), (1,)), ((), ())),
                         preferred_element_type=jnp.float32)
    tr_ref[...] = tr + b_ref[...]
    tc_ref[...] = lax.dot_general(col_ref[...], w[:, HALF:],
                                  (((1,), (1,)), ((), ())),
                                  preferred_element_type=jnp.float32)


def _make_tables(row_embed, col_embed, w, b):
    return pl.pallas_call(
        _tables_body,
        out_shape=(
            jax.ShapeDtypeStruct((GRID, D_OUT), jnp.float32),
            jax.ShapeDtypeStruct((GRID, D_OUT), jnp.float32),
        ),
    )(row_embed, col_embed, w, b.reshape(1, D_OUT))


_sc_mesh = plsc.VectorSubcoreMesh(core_axis_name="c", subcore_axis_name="s")


@functools.partial(
    pl.kernel,
    out_type=jax.ShapeDtypeStruct((SPOTS, D_OUT), jnp.float32),
    mesh=_sc_mesh,
    scratch_types=[
        pltpu.VMEM((NCHUNK, CHUNK), jnp.int32),
        pltpu.VMEM((NCHUNK, CHUNK), jnp.int32),
        pltpu.VMEM((CHUNK, D_OUT), jnp.float32),
        pltpu.VMEM((CHUNK, D_OUT), jnp.float32),
        pltpu.VMEM((CHUNK, D_OUT), jnp.float32),
        pltpu.VMEM((CHUNK, D_OUT), jnp.float32),
        pltpu.VMEM((CHUNK, D_OUT), jnp.float32),
        pltpu.VMEM((CHUNK, D_OUT), jnp.float32),
        pltpu.VMEM_SHARED((GRID, D_OUT), jnp.float32),
        pltpu.VMEM_SHARED((GRID, D_OUT), jnp.float32),
        pltpu.SemaphoreType.DMA,
        pltpu.SemaphoreType.DMA,
        pltpu.SemaphoreType.DMA,
        pltpu.SemaphoreType.DMA,
        pltpu.SemaphoreType.DMA,
        pltpu.SemaphoreType.DMA,
    ],
)
def _sc_lookup(tr_hbm, tc_hbm, rows_hbm, cols_hbm, out_hbm,
               idxr, idxc, bufr0, bufr1, bufc0, bufc1, bufo0, bufo1,
               tr_sp, tc_sp, semr0, semr1, semc0, semc1, semo0, semo1):
    wid = lax.axis_index("s") * NC + lax.axis_index("c")
    base = wid * PER_W
    bufr = (bufr0, bufr1)
    bufc = (bufc0, bufc1)
    bufo = (bufo0, bufo1)
    semr = (semr0, semr1)
    semc = (semc0, semc1)
    semo = (semo0, semo1)
    @pl.when(lax.axis_index("s") == 0)
    def _stage():
        pltpu.sync_copy(tr_hbm, tr_sp)
        pltpu.sync_copy(tc_hbm, tc_sp)

    pltpu.sync_copy(rows_hbm.at[wid], idxr)
    pltpu.sync_copy(cols_hbm.at[wid], idxc)
    plsc.subcore_barrier()
    gr = [pltpu.async_copy(tr_sp.at[idxr.at[p]], bufr[p], semr[p])
          for p in range(2)]
    gc = [pltpu.async_copy(tc_sp.at[idxc.at[p]], bufc[p], semc[p])
          for p in range(2)]
    oc = [None, None]
    for j in range(NCHUNK):
        p = j & 1
        gr[p].wait()
        gc[p].wait()
        if oc[p] is not None:
            oc[p].wait()

        def _add_row(r, _, p=p):
            for c in range(D_OUT // 16):
                sl = pl.ds(c * 16, 16)
                bufo[p][r, sl] = bufr[p][r, sl] + bufc[p][r, sl]
            return 0

        lax.fori_loop(0, CHUNK, _add_row, 0)
        oc[p] = pltpu.async_copy(
            bufo[p], out_hbm.at[pl.ds(base + j * CHUNK, CHUNK)], semo[p])
        if j + 2 < NCHUNK:
            gr[p] = pltpu.async_copy(tr_sp.at[idxr.at[j + 2]], bufr[p], semr[p])
            gc[p] = pltpu.async_copy(tc_sp.at[idxc.at[j + 2]], bufc[p], semc[p])
    oc[0].wait()
    oc[1].wait()


def kernel(coords, row_embed, col_embed, W, b):
    batch, n_spots, _ = coords.shape
    tr, tc = _make_tables(row_embed, col_embed, W, b)
    cc = jnp.clip(coords.astype(jnp.int32), 0, GRID - 1)
    rows = cc[..., 0].reshape(NW, NCHUNK, CHUNK)
    cols = cc[..., 1].reshape(NW, NCHUNK, CHUNK)
    out = _sc_lookup(tr, tc, rows, cols)
    return out.reshape(batch, n_spots, D_OUT)
